# Initial kernel scaffold; baseline (speedup 1.0000x reference)
import jax
import jax.numpy as jnp
from jax.experimental import pallas as pl


def kernel(x, table):
    raise NotImplementedError("write your pallas kernel here")


# speedup vs baseline: 1.8438x; 1.8438x over previous
import functools

import jax
import jax.numpy as jnp
from jax import lax
from jax.experimental import pallas as pl
from jax.experimental.pallas import tpu as pltpu
from jax.experimental.pallas import tpu_sc as plsc

EMBED = 64
ROWS = 16384
COLS = 50
B_TOTAL = ROWS * COLS
NC = 2
NS = 16
NW = NC * NS
B_PER_W = B_TOTAL // NW

IW = 128
K = 8
CHUNK = IW * K
NCHUNK = B_PER_W // CHUNK

_mesh = plsc.VectorSubcoreMesh(core_axis_name="c", subcore_axis_name="s")


@functools.partial(
    pl.kernel,
    mesh=_mesh,
    out_type=jax.ShapeDtypeStruct((B_TOTAL, EMBED), jnp.float32),
    scratch_types=[
        pltpu.VMEM((K, IW), jnp.int32),
        pltpu.VMEM((CHUNK, EMBED), jnp.float32),
        pltpu.SemaphoreType.DMA,
    ],
    compiler_params=pltpu.CompilerParams(use_tc_tiling_on_sc=False),
)
def _embed_sc(idx_hbm, table_hbm, out_hbm, idx_v, rows_v, sem):
    wid = lax.axis_index("s") * NC + lax.axis_index("c")
    base = wid * B_PER_W

    def body(c, carry):
        off = pl.multiple_of(base + c * CHUNK, CHUNK)
        irow = pl.multiple_of(off // IW, K)
        pltpu.sync_copy(idx_hbm.at[pl.ds(irow, K)], idx_v)
        copies = [
            pltpu.async_copy(
                table_hbm.at[idx_v.at[j]],
                rows_v.at[pl.ds(j * IW, IW)],
                sem,
            )
            for j in range(K)
        ]
        for cp in copies:
            cp.wait()
        pltpu.sync_copy(rows_v, out_hbm.at[pl.ds(off, CHUNK)])
        return carry

    lax.fori_loop(0, NCHUNK, body, 0)


def kernel(x, table):
    idx = x.reshape(B_TOTAL // IW, IW).astype(jnp.int32)
    out = _embed_sc(idx, table)
    return out.reshape(ROWS, COLS, EMBED)


# speedup vs baseline: 1.8904x; 1.0253x over previous
import functools

import jax
import jax.numpy as jnp
from jax import lax
from jax.experimental import pallas as pl
from jax.experimental.pallas import tpu as pltpu
from jax.experimental.pallas import tpu_sc as plsc

EMBED = 64
ROWS = 16384
COLS = 50
B_TOTAL = ROWS * COLS
NC = 2
NS = 16
NW = NC * NS
B_PER_W = B_TOTAL // NW

IW = 64
K = 8
CHUNK = IW * K
NCHUNK = B_PER_W // CHUNK
NPAIR = NCHUNK // 2

_mesh = plsc.VectorSubcoreMesh(core_axis_name="c", subcore_axis_name="s")


@functools.partial(
    pl.kernel,
    mesh=_mesh,
    out_type=jax.ShapeDtypeStruct((B_TOTAL, EMBED), jnp.float32),
    scratch_types=[
        pltpu.VMEM((2, K, IW), jnp.int32),
        pltpu.VMEM((2, CHUNK, EMBED), jnp.float32),
        pltpu.SemaphoreType.DMA,
        pltpu.SemaphoreType.DMA,
        pltpu.SemaphoreType.DMA,
        pltpu.SemaphoreType.DMA,
    ],
    compiler_params=pltpu.CompilerParams(use_tc_tiling_on_sc=False),
)
def _embed_sc(idx_hbm, table_hbm, out_hbm, idx_v, rows_v, sem_g0, sem_g1,
              sem_w0, sem_w1):
    wid = lax.axis_index("s") * NC + lax.axis_index("c")
    base = wid * B_PER_W
    sem_g = (sem_g0, sem_g1)
    sem_w = (sem_w0, sem_w1)

    def load_idx(c, b):
        irow = pl.multiple_of((base + c * CHUNK) // IW, K)
        pltpu.sync_copy(idx_hbm.at[pl.ds(irow, K)], idx_v.at[b])

    def fire_gathers(b):
        for j in range(K):
            pltpu.async_copy(
                table_hbm.at[idx_v.at[b, j]],
                rows_v.at[b, pl.ds(j * IW, IW)],
                sem_g[b],
            )

    def wait_gathers(b):
        for j in range(K):
            pltpu.make_async_copy(
                table_hbm.at[idx_v.at[b, j]],
                rows_v.at[b, pl.ds(j * IW, IW)],
                sem_g[b],
            ).wait()

    def fire_writeback(c, b):
        off = pl.multiple_of(base + c * CHUNK, CHUNK)
        pltpu.async_copy(rows_v.at[b], out_hbm.at[pl.ds(off, CHUNK)], sem_w[b])

    def wait_writeback(b):
        pltpu.make_async_copy(
            rows_v.at[b], out_hbm.at[pl.ds(0, CHUNK)], sem_w[b]
        ).wait()

    load_idx(0, 0)
    fire_gathers(0)

    def body(g, carry):
        c0 = g * 2
        c1 = c0 + 1
        load_idx(c1, 1)

        @pl.when(g > 0)
        def _():
            wait_writeback(1)

        fire_gathers(1)
        wait_gathers(0)
        fire_writeback(c0, 0)
        @pl.when(g < NPAIR - 1)
        def _():
            load_idx(c0 + 2, 0)
            wait_writeback(0)
            fire_gathers(0)

        wait_gathers(1)
        fire_writeback(c1, 1)
        return carry

    lax.fori_loop(0, NPAIR, body, 0)
    wait_writeback(0)
    wait_writeback(1)


def kernel(x, table):
    idx = x.reshape(B_TOTAL // IW, IW).astype(jnp.int32)
    out = _embed_sc(idx, table)
    return out.reshape(ROWS, COLS, EMBED)
